# baseline (speedup 1.0000x reference)
import jax
import jax.numpy as jnp
from jax.experimental import pallas as pl


def _fused_step(
    x_ref, adj_ref, cand_ref, mask_ref, hgm_ref, pool_ref,
    g0w1_ref, g0b1_ref, g0w2_ref, g0b2_ref,
    g1w1_ref, g1b1_ref, g1w2_ref, g1b2_ref,
    g2w1_ref, g2b1_ref, g2w2_ref, g2b2_ref,
    aw1a_ref, aw1b_ref, aw1c_ref, ab1_ref,
    aw2_ref, ab2_ref, aw3_ref, ab3_ref,
    cw1_ref, cb1_ref, cw2_ref, cb2_ref, cw3_ref, cb3_ref,
    ti_ref, ai_ref, la_ref, pr_ref, hg_ref, jv_ref,
):
    f32 = jnp.float32
    G, T, _ = adj_ref.shape
    gin = ((g0w1_ref, g0b1_ref, g0w2_ref, g0b2_ref),
           (g1w1_ref, g1b1_ref, g1w2_ref, g1b2_ref),
           (g2w1_ref, g2b1_ref, g2w2_ref, g2b2_ref))

    R = range(G)
    dot = lambda p, q: jnp.dot(p, q, preferred_element_type=f32)
    A = [adj_ref[g] for g in R]
    h = [x_ref[pl.ds(g * T, T), :] for g in R]

    for w1_ref, b1_ref, w2_ref, b2_ref in gin:
        w1, b1, w2, b2 = w1_ref[...], b1_ref[...], w2_ref[...], b2_ref[...]
        neigh = [dot(A[g], h[g]) for g in R]
        pooled = [neigh[g] + h[g] for g in R]
        m = [jnp.maximum(dot(pooled[g], w1) + b1, 0.0) for g in R]
        m = [dot(m[g], w2) + b2 for g in R]
        h = [jnp.maximum(m[g], 0.0) for g in R]

    pool = pool_ref[...]
    hg = [dot(pool, h[g]) for g in R]

    cand = [cand_ref[g] for g in R]
    nj = cand[0].shape[0]
    iota_t = jax.lax.broadcasted_iota(jnp.int32, (nj, T), 1)
    onehot = [(iota_t == cand[g]).astype(f32) for g in R]
    cf = [dot(onehot[g], h[g]) for g in R]

    row = [(dot(hg[g], aw1b_ref[...])
            + dot(hgm_ref[g], aw1c_ref[...])
            + ab1_ref[...]) for g in R]
    t1 = [jnp.tanh(dot(cf[g], aw1a_ref[...]) + row[g]) for g in R]
    t2 = [jnp.tanh(dot(t1[g], aw2_ref[...]) + ab2_ref[...]) for g in R]
    sc = [dot(t2[g], aw3_ref[...]) + ab3_ref[...] for g in R]
    sc = [sc[g] - mask_ref[g] * 1e30 for g in R]

    smax = [jnp.max(sc[g], axis=0, keepdims=True) for g in R]
    e = [jnp.exp(sc[g] - smax[g]) for g in R]
    esum = [jnp.sum(e[g], axis=0, keepdims=True) for g in R]
    prob = [e[g] / esum[g] for g in R]

    iota_nj = jax.lax.broadcasted_iota(jnp.int32, (nj, 1), 0)
    pmax = [jnp.max(prob[g], axis=0, keepdims=True) for g in R]
    am = [jnp.min(jnp.where(prob[g] == pmax[g], iota_nj, nj),
                  axis=0, keepdims=True) for g in R]
    task = [jnp.sum(jnp.where(iota_nj == am[g], cand[g], 0),
                    axis=0, keepdims=True) for g in R]
    la = [jnp.log(pmax[g] + 1e-10) for g in R]

    v1 = [jnp.tanh(dot(hg[g], cw1_ref[...]) + cb1_ref[...]) for g in R]
    v2 = [jnp.tanh(dot(v1[g], cw2_ref[...]) + cb2_ref[...]) for g in R]
    jv = [dot(v2[g], cw3_ref[...]) + cb3_ref[...] for g in R]

    for g in R:
        ti_ref[g] = task[g].reshape(1, 1)
        ai_ref[g] = am[g].reshape(1, 1)
        la_ref[g] = la[g].reshape(1, 1)
        pr_ref[g] = prob[g].reshape(nj, 1)
        hg_ref[g] = hg[g].reshape(1, -1)
        jv_ref[g] = jv[g].reshape(1, -1)


def kernel(x_fea, graph_pool_avg, padded_nei, adj, candidate, h_g_m_pooled,
           mask_operation,
           gin0_w1, gin0_b1, gin0_w2, gin0_b2,
           gin1_w1, gin1_b1, gin1_w2, gin1_b2,
           gin2_w1, gin2_b1, gin2_w2, gin2_b2,
           a_w1, a_b1, a_w2, a_b2, a_w3, a_b3,
           c_w1, c_b1, c_w2, c_b2, c_w3, c_b3):
    B, T, _ = adj.shape
    NJ = candidate.shape[1]
    DIN = x_fea.shape[1]
    H = gin0_w2.shape[0]
    G = 4
    f32 = jnp.float32

    cand3 = candidate.astype(jnp.int32).reshape(B, NJ, 1)
    mask3 = mask_operation.astype(f32).reshape(B, NJ, 1)
    hgm3 = h_g_m_pooled.reshape(B, 1, H)
    pool_row = jnp.full((1, T), 1.0 / T, f32)

    aw1a, aw1b, aw1c = a_w1[:H], a_w1[H:2 * H], a_w1[2 * H:]
    r2 = lambda v: v.reshape(1, -1)

    def full(w):
        nd = w.ndim
        return pl.BlockSpec(w.shape, lambda b, _n=nd: (0,) * _n)

    weights = (gin0_w1, r2(gin0_b1), gin0_w2, r2(gin0_b2),
               gin1_w1, r2(gin1_b1), gin1_w2, r2(gin1_b2),
               gin2_w1, r2(gin2_b1), gin2_w2, r2(gin2_b2),
               aw1a, aw1b, aw1c, r2(a_b1),
               a_w2, r2(a_b2), a_w3, r2(a_b3),
               c_w1, r2(c_b1), c_w2, r2(c_b2), c_w3, r2(c_b3))

    in_specs = [
        pl.BlockSpec((G * T, DIN), lambda b: (b, 0)),
        pl.BlockSpec((G, T, T), lambda b: (b, 0, 0)),
        pl.BlockSpec((G, NJ, 1), lambda b: (b, 0, 0)),
        pl.BlockSpec((G, NJ, 1), lambda b: (b, 0, 0)),
        pl.BlockSpec((G, 1, H), lambda b: (b, 0, 0)),
        full(pool_row),
    ] + [full(w) for w in weights]

    out_shapes = (
        jax.ShapeDtypeStruct((B, 1, 1), jnp.int32),
        jax.ShapeDtypeStruct((B, 1, 1), jnp.int32),
        jax.ShapeDtypeStruct((B, 1, 1), f32),
        jax.ShapeDtypeStruct((B, NJ, 1), f32),
        jax.ShapeDtypeStruct((B, 1, H), f32),
        jax.ShapeDtypeStruct((B, 1, 4), f32),
    )
    out_specs = (
        pl.BlockSpec((G, 1, 1), lambda b: (b, 0, 0)),
        pl.BlockSpec((G, 1, 1), lambda b: (b, 0, 0)),
        pl.BlockSpec((G, 1, 1), lambda b: (b, 0, 0)),
        pl.BlockSpec((G, NJ, 1), lambda b: (b, 0, 0)),
        pl.BlockSpec((G, 1, H), lambda b: (b, 0, 0)),
        pl.BlockSpec((G, 1, 4), lambda b: (b, 0, 0)),
    )

    ti, ai, la, pr, hg, jv = pl.pallas_call(
        _fused_step,
        grid=(B // G,),
        in_specs=in_specs,
        out_specs=out_specs,
        out_shape=out_shapes,
    )(x_fea, adj, cand3, mask3, hgm3, pool_row, *weights)

    return (ti.reshape(B), ai.reshape(B), la.reshape(B),
            pr.reshape(B, NJ), hg.reshape(B, H), jv.reshape(B, 4))


# speedup vs baseline: 1.1049x; 1.1049x over previous
import jax
import jax.numpy as jnp
from jax.experimental import pallas as pl

_G = 2


def _fused_step(
    x_ref, adj_ref, cand_ref, mask_ref, hgm_ref,
    g0w1_ref, g0b1_ref, g0w2_ref, g0b2_ref,
    g1w1_ref, g1b1_ref, g1w2_ref, g1b2_ref,
    g2w1_ref, g2b1_ref, g2w2_ref, g2b2_ref,
    aw1_ref, ab1_ref, aw2_ref, ab2_ref, aw3_ref, ab3_ref,
    cw1_ref, cb1_ref, cw2_ref, cb2_ref, cw3_ref, cb3_ref,
    ti_ref, ai_ref, la_ref, pr_ref, hg_ref, jv_ref,
):
    f32 = jnp.float32
    G, T, _ = adj_ref.shape
    H = g0w2_ref.shape[0]
    nj = cand_ref.shape[1]
    step = pl.program_id(0)
    rows = [step * G + g for g in range(G)]
    R = range(G)
    dot = lambda p, q: jnp.dot(p, q, preferred_element_type=f32)
    gin = ((g0w1_ref, g0b1_ref, g0w2_ref, g0b2_ref),
           (g1w1_ref, g1b1_ref, g1w2_ref, g1b2_ref),
           (g2w1_ref, g2b1_ref, g2w2_ref, g2b2_ref))

    A = [adj_ref[g] for g in R]
    h = [x_ref[pl.ds(g * T, T), :] for g in R]

    for w1_ref, b1_ref, w2_ref, b2_ref in gin:
        w1, b1, w2, b2 = w1_ref[...], b1_ref[...], w2_ref[...], b2_ref[...]
        neigh = [dot(A[g], h[g]) for g in R]
        pooled = [neigh[g] + h[g] for g in R]
        m = [jnp.maximum(dot(pooled[g], w1) + b1, 0.0) for g in R]
        m = [dot(m[g], w2) + b2 for g in R]
        h = [jnp.maximum(m[g], 0.0) for g in R]

    pool = jnp.full((1, T), 1.0 / T, f32)
    hg = [dot(pool, h[g]) for g in R]

    cand_row = [cand_ref[pl.ds(rows[g], 1), :] for g in R]
    cand_col = [jnp.transpose(cand_row[g]) for g in R]
    iota_t = jax.lax.broadcasted_iota(jnp.int32, (nj, T), 1)
    onehot = [(iota_t == cand_col[g]).astype(f32) for g in R]
    cf = [dot(onehot[g], h[g]) for g in R]

    aw1a = aw1_ref[0:H, :]
    aw1b = aw1_ref[H:2 * H, :]
    aw1c = aw1_ref[2 * H:3 * H, :]
    hgm = [hgm_ref[pl.ds(rows[g], 1), :] for g in R]
    row = [dot(hg[g], aw1b) + dot(hgm[g], aw1c) + ab1_ref[...] for g in R]
    t1 = [jnp.tanh(dot(cf[g], aw1a) + row[g]) for g in R]
    t2 = [jnp.tanh(dot(t1[g], aw2_ref[...]) + ab2_ref[...]) for g in R]
    sc = [dot(t2[g], aw3_ref[...]) + ab3_ref[...] for g in R]
    scr = [jnp.transpose(sc[g]) - mask_ref[pl.ds(rows[g], 1), :] * 1e30
           for g in R]

    smax = [jnp.max(scr[g], axis=1, keepdims=True) for g in R]
    e = [jnp.exp(scr[g] - smax[g]) for g in R]
    esum = [jnp.sum(e[g], axis=1, keepdims=True) for g in R]
    prob = [e[g] / esum[g] for g in R]

    iota_nj = jax.lax.broadcasted_iota(jnp.int32, (1, nj), 1)
    pmax = [jnp.max(prob[g], axis=1, keepdims=True) for g in R]
    am = [jnp.min(jnp.where(prob[g] == pmax[g], iota_nj, nj),
                  axis=1, keepdims=True) for g in R]
    task = [jnp.sum(jnp.where(iota_nj == am[g], cand_row[g], 0),
                    axis=1, keepdims=True) for g in R]
    la = [jnp.log(pmax[g] + 1e-10) for g in R]

    v1 = [jnp.tanh(dot(hg[g], cw1_ref[...]) + cb1_ref[...]) for g in R]
    v2 = [jnp.tanh(dot(v1[g], cw2_ref[...]) + cb2_ref[...]) for g in R]
    jv = [dot(v2[g], cw3_ref[...]) + cb3_ref[...] for g in R]

    for g in R:
        r = pl.ds(rows[g], 1)
        ti_ref[r, :] = task[g]
        ai_ref[r, :] = am[g]
        la_ref[r, :] = la[g]
        pr_ref[r, :] = prob[g]
        hg_ref[r, :] = hg[g]
        jv_ref[r, :] = jv[g]


def kernel(x_fea, graph_pool_avg, padded_nei, adj, candidate, h_g_m_pooled,
           mask_operation,
           gin0_w1, gin0_b1, gin0_w2, gin0_b2,
           gin1_w1, gin1_b1, gin1_w2, gin1_b2,
           gin2_w1, gin2_b1, gin2_w2, gin2_b2,
           a_w1, a_b1, a_w2, a_b2, a_w3, a_b3,
           c_w1, c_b1, c_w2, c_b2, c_w3, c_b3):
    B, T, _ = adj.shape
    NJ = candidate.shape[1]
    DIN = x_fea.shape[1]
    H = gin0_w2.shape[0]
    G = _G
    f32 = jnp.float32

    cand = candidate.astype(jnp.int32)
    mask_f = mask_operation.astype(f32)
    r2 = lambda v: v.reshape(1, -1)

    def full(w):
        nd = w.ndim
        return pl.BlockSpec(w.shape, lambda b, _n=nd: (0,) * _n)

    weights = (gin0_w1, r2(gin0_b1), gin0_w2, r2(gin0_b2),
               gin1_w1, r2(gin1_b1), gin1_w2, r2(gin1_b2),
               gin2_w1, r2(gin2_b1), gin2_w2, r2(gin2_b2),
               a_w1, r2(a_b1), a_w2, r2(a_b2), a_w3, r2(a_b3),
               c_w1, r2(c_b1), c_w2, r2(c_b2), c_w3, r2(c_b3))

    in_specs = [
        pl.BlockSpec((G * T, DIN), lambda b: (b, 0)),
        pl.BlockSpec((G, T, T), lambda b: (b, 0, 0)),
        full(cand),
        full(mask_f),
        full(h_g_m_pooled),
    ] + [full(w) for w in weights]

    out_shapes = (
        jax.ShapeDtypeStruct((B, 1), jnp.int32),
        jax.ShapeDtypeStruct((B, 1), jnp.int32),
        jax.ShapeDtypeStruct((B, 1), f32),
        jax.ShapeDtypeStruct((B, NJ), f32),
        jax.ShapeDtypeStruct((B, H), f32),
        jax.ShapeDtypeStruct((B, 4), f32),
    )
    out_specs = tuple(full(s) for s in out_shapes)

    ti, ai, la, pr, hg, jv = pl.pallas_call(
        _fused_step,
        grid=(B // G,),
        in_specs=in_specs,
        out_specs=out_specs,
        out_shape=out_shapes,
    )(x_fea, adj, cand, mask_f, h_g_m_pooled, *weights)

    return (ti.reshape(B), ai.reshape(B), la.reshape(B), pr, hg, jv)
